# Initial kernel scaffold; baseline (speedup 1.0000x reference)
import jax
import jax.numpy as jnp
from jax.experimental import pallas as pl


def kernel(walks, u_table, v_table, idx_pos_u, idx_pos_v, idx_neg_u, idx_neg_v):
    raise NotImplementedError("write your pallas kernel here")


# speedup vs baseline: 1.9915x; 1.9915x over previous
import functools

import jax
import jax.numpy as jnp
from jax import lax
from jax.experimental import pallas as pl
from jax.experimental.pallas import tpu as pltpu
from jax.experimental.pallas import tpu_sc as plsc

NC = 2
NS = 16
NW = NC * NS
L = 16
C = 128


def _sc_pair_scores(nodes, u_table, v_table, a_idx, b_idx):
    n_pairs = a_idx.shape[0]
    n_chunks = n_pairs // C
    iters = (n_chunks + NW - 1) // NW

    mesh = plsc.VectorSubcoreMesh(core_axis_name="c", subcore_axis_name="s")

    @functools.partial(
        pl.kernel,
        out_type=jax.ShapeDtypeStruct((n_pairs,), jnp.float32),
        mesh=mesh,
        compiler_params=pltpu.CompilerParams(needs_layout_passes=False),
        scratch_types=[
            pltpu.VMEM((nodes.shape[0],), jnp.int32),
            pltpu.VMEM((C,), jnp.int32),
            pltpu.VMEM((C,), jnp.int32),
            pltpu.VMEM((C,), jnp.int32),
            pltpu.VMEM((C,), jnp.int32),
            pltpu.VMEM((C, 128), jnp.float32),
            pltpu.VMEM((C, 128), jnp.float32),
            pltpu.VMEM((C,), jnp.float32),
            pltpu.SemaphoreType.DMA,
            pltpu.SemaphoreType.DMA,
        ],
    )
    def k(nodes_hbm, u_hbm, v_hbm, a_hbm, b_hbm, out_hbm,
          nodes_v, a_v, b_v, ga_v, gb_v, urows_v, vrows_v, out_v,
          sem_u, sem_v):
        wid = lax.axis_index("s") * NC + lax.axis_index("c")
        pltpu.sync_copy(nodes_hbm, nodes_v)

        def chunk_body(t, carry):
            c = wid + t * NW

            @pl.when(c < n_chunks)
            def _():
                base = c * C
                pltpu.sync_copy(a_hbm.at[pl.ds(base, C)], a_v)
                pltpu.sync_copy(b_hbm.at[pl.ds(base, C)], b_v)

                def compose(i, carry2):
                    ai = a_v[pl.ds(i * L, L)]
                    bi = b_v[pl.ds(i * L, L)]
                    ga_v[pl.ds(i * L, L)] = plsc.load_gather(nodes_v, [ai])
                    gb_v[pl.ds(i * L, L)] = plsc.load_gather(nodes_v, [bi])
                    return carry2

                lax.fori_loop(0, C // L, compose, 0)

                cu = pltpu.async_copy(u_hbm.at[ga_v], urows_v, sem_u)
                cv = pltpu.async_copy(v_hbm.at[gb_v], vrows_v, sem_v)
                cu.wait()
                cv.wait()

                lanes = lax.iota(jnp.int32, L)

                def dot_group(g, carry2):
                    rb = g * L
                    scores = jnp.zeros((L,), jnp.float32)
                    for j in range(L):
                        r = rb + j
                        p = urows_v[r, pl.ds(0, L)] * vrows_v[r, pl.ds(0, L)]
                        for kk in range(1, 128 // L):
                            p = p + (urows_v[r, pl.ds(kk * L, L)]
                                     * vrows_v[r, pl.ds(kk * L, L)])
                        tot = jnp.broadcast_to(jnp.sum(p), (L,))
                        scores = jnp.where(lanes == j, tot, scores)
                    scores = jnp.clip(scores, -6.01, 6.01)
                    out_v[pl.ds(rb, L)] = 1.0 / (1.0 + jnp.exp(-scores))
                    return carry2

                lax.fori_loop(0, C // L, dot_group, 0)

                pltpu.sync_copy(out_v, out_hbm.at[pl.ds(base, C)])

            return carry

        lax.fori_loop(0, iters, chunk_body, 0)

    return k(nodes, u_table, v_table, a_idx, b_idx)


def kernel(walks, u_table, v_table, idx_pos_u, idx_pos_v, idx_neg_u, idx_neg_v):
    nodes = walks.reshape(-1)
    a_idx = jnp.concatenate([idx_pos_u, idx_neg_u])
    b_idx = jnp.concatenate([idx_pos_v, idx_neg_v])
    return _sc_pair_scores(nodes, u_table, v_table, a_idx, b_idx)


# speedup vs baseline: 4.4925x; 2.2559x over previous
import functools

import jax
import jax.numpy as jnp
from jax import lax
from jax.experimental import pallas as pl
from jax.experimental.pallas import tpu as pltpu
from jax.experimental.pallas import tpu_sc as plsc

NC = 2
NS = 16
NW = NC * NS
L = 16
C = 128
D = 128
RK = D // L


def _sc_pair_scores(nodes, u_table, v_table, ab):
    n_chunks = ab.shape[0]
    n_pairs = n_chunks * C
    iters = (n_chunks + NW - 1) // NW

    mesh = plsc.VectorSubcoreMesh(core_axis_name="c", subcore_axis_name="s")

    @functools.partial(
        pl.kernel,
        out_type=jax.ShapeDtypeStruct((n_pairs,), jnp.float32),
        mesh=mesh,
        compiler_params=pltpu.CompilerParams(needs_layout_passes=False),
        scratch_types=[
            pltpu.VMEM((nodes.shape[0],), jnp.int32),
            pltpu.VMEM((2, 2, C), jnp.int32),
            pltpu.VMEM((2, 2, C), jnp.int32),
            pltpu.VMEM((2, C, D), jnp.float32),
            pltpu.VMEM((2, C, D), jnp.float32),
            pltpu.VMEM((L, L), jnp.float32),
            pltpu.VMEM((2, C), jnp.float32),
            pltpu.SemaphoreType.DMA,
            pltpu.SemaphoreType.DMA,
            pltpu.SemaphoreType.DMA,
            pltpu.SemaphoreType.DMA,
            pltpu.SemaphoreType.DMA,
            pltpu.SemaphoreType.DMA,
            pltpu.SemaphoreType.DMA,
            pltpu.SemaphoreType.DMA,
        ],
    )
    def k(nodes_hbm, u_hbm, v_hbm, ab_hbm, out_hbm,
          nodes_v, iab_v, gab_v, urows_v, vrows_v, pt_v, out_v,
          sem_i0, sem_i1, sem_u0, sem_u1, sem_v0, sem_v1, sem_o0, sem_o1):
        wid = lax.axis_index("s") * NC + lax.axis_index("c")
        pltpu.sync_copy(nodes_hbm, nodes_v)

        sem_i = (sem_i0, sem_i1)
        sem_u = (sem_u0, sem_u1)
        sem_v = (sem_v0, sem_v1)
        sem_o = (sem_o0, sem_o1)

        def chunk_at(kk):
            return wid + kk * NW

        def issue_idx(c, s):
            pltpu.async_copy(ab_hbm.at[c], iab_v.at[s], sem_i[s])

        def wait_idx(c, s):
            pltpu.make_async_copy(ab_hbm.at[c], iab_v.at[s], sem_i[s]).wait()

        def compose_and_issue_rows(s):
            def compose(i, cr):
                ai = iab_v[s, 0, pl.ds(i * L, L)]
                bi = iab_v[s, 1, pl.ds(i * L, L)]
                gab_v[s, 0, pl.ds(i * L, L)] = plsc.load_gather(nodes_v, [ai])
                gab_v[s, 1, pl.ds(i * L, L)] = plsc.load_gather(nodes_v, [bi])
                return cr

            lax.fori_loop(0, C // L, compose, 0)
            pltpu.async_copy(u_hbm.at[gab_v.at[s, 0]], urows_v.at[s], sem_u[s])
            pltpu.async_copy(v_hbm.at[gab_v.at[s, 1]], vrows_v.at[s], sem_v[s])

        def wait_rows(s):
            pltpu.make_async_copy(
                u_hbm.at[gab_v.at[s, 0]], urows_v.at[s], sem_u[s]).wait()
            pltpu.make_async_copy(
                v_hbm.at[gab_v.at[s, 1]], vrows_v.at[s], sem_v[s]).wait()

        def wait_out(s):
            pltpu.make_async_copy(
                out_v.at[s], out_hbm.at[pl.ds(0, C)], sem_o[s]).wait()

        lanes = lax.iota(jnp.int32, L)

        def compute(s):
            def dot_group(g, cr):
                rb = g * L
                for j in range(L):
                    r = rb + j
                    p = urows_v[s, r, pl.ds(0, L)] * vrows_v[s, r, pl.ds(0, L)]
                    for kk in range(1, RK):
                        p = p + (urows_v[s, r, pl.ds(kk * L, L)]
                                 * vrows_v[s, r, pl.ds(kk * L, L)])
                    pt_v[j, pl.ds(0, L)] = p
                sc = plsc.load_gather(pt_v, [lanes, jnp.zeros((L,), jnp.int32)])
                for c2 in range(1, L):
                    sc = sc + plsc.load_gather(
                        pt_v, [lanes, jnp.full((L,), c2, jnp.int32)])
                sc = jnp.clip(sc, -6.01, 6.01)
                out_v[s, pl.ds(rb, L)] = 1.0 / (1.0 + jnp.exp(-sc))
                return cr

            lax.fori_loop(0, C // L, dot_group, 0)

        def half(kk, s):
            c0 = chunk_at(kk)
            c1 = chunk_at(kk + 1)
            c2 = chunk_at(kk + 2)

            @pl.when(c2 < n_chunks)
            def _():
                issue_idx(c2, s)

            @pl.when(c1 < n_chunks)
            def _():
                wait_idx(c1, 1 - s)
                compose_and_issue_rows(1 - s)

            @pl.when(c0 < n_chunks)
            def _():
                wait_rows(s)

                @pl.when(kk >= 2)
                def _():
                    wait_out(s)

                compute(s)
                pltpu.async_copy(
                    out_v.at[s], out_hbm.at[pl.ds(c0 * C, C)], sem_o[s])

        pltpu.sync_copy(ab_hbm.at[chunk_at(0)], iab_v.at[0])
        compose_and_issue_rows(0)

        @pl.when(chunk_at(1) < n_chunks)
        def _():
            issue_idx(chunk_at(1), 1)

        def body(u, cr):
            half(2 * u, 0)
            half(2 * u + 1, 1)
            return cr

        lax.fori_loop(0, (iters + 1) // 2, body, 0)

        n_my = (n_chunks - wid + NW - 1) // NW

        @pl.when(lax.rem(n_my - 1, 2) == 0)
        def _():
            wait_out(0)

            @pl.when(n_my >= 2)
            def _():
                wait_out(1)

        @pl.when(lax.rem(n_my - 1, 2) == 1)
        def _():
            wait_out(1)

            @pl.when(n_my >= 2)
            def _():
                wait_out(0)

    return k(nodes, u_table, v_table, ab)


def kernel(walks, u_table, v_table, idx_pos_u, idx_pos_v, idx_neg_u, idx_neg_v):
    nodes = walks.reshape(-1)
    a_idx = jnp.concatenate([idx_pos_u, idx_neg_u])
    b_idx = jnp.concatenate([idx_pos_v, idx_neg_v])
    ab = jnp.stack([a_idx.reshape(-1, C), b_idx.reshape(-1, C)], axis=1)
    return _sc_pair_scores(nodes, u_table, v_table, ab)
